# baseline (speedup 1.0000x reference)
import functools

import jax
import jax.numpy as jnp
from jax import lax
from jax.experimental import pallas as pl
from jax.experimental.pallas import tpu as pltpu
from jax.experimental.pallas import tpu_sc as plsc


def _sc_gather(x, embed_table):
    B = x.shape[0]
    H = embed_table.shape[1]
    info = plsc.get_sparse_core_info()
    NW = info.num_cores * info.num_subcores
    assert B % (8 * NW) == 0
    b_per_w = B // NW
    mesh = plsc.VectorSubcoreMesh(core_axis_name="c", subcore_axis_name="s")

    @functools.partial(
        pl.kernel,
        mesh=mesh,
        out_type=jax.ShapeDtypeStruct((B, H), jnp.float32),
        scratch_types=[
            pltpu.VMEM((b_per_w,), jnp.int32),
            pltpu.VMEM((b_per_w, H), jnp.float32),
            pltpu.SemaphoreType.DMA,
        ],
        compiler_params=pltpu.CompilerParams(use_tc_tiling_on_sc=False),
    )
    def gather_kernel(idx_hbm, table_hbm, out_hbm, idx_v, rows_v, sem):
        wid = lax.axis_index("s") * info.num_cores + lax.axis_index("c")
        base = wid * b_per_w
        pltpu.sync_copy(idx_hbm.at[pl.ds(base, b_per_w)], idx_v)
        pltpu.async_copy(table_hbm.at[idx_v], rows_v, sem).wait()
        pltpu.sync_copy(rows_v, out_hbm.at[pl.ds(base, b_per_w)])

    return gather_kernel(x, embed_table)


def _head_matmul(gathered, head_w, head_b, vb, nbuf):
    B, H = gathered.shape
    V = head_w.shape[0]
    assert vb % 128 == 0
    nsteps = pl.cdiv(V, vb)
    vb_last = V - (nsteps - 1) * vb

    def body(g_ref, w_ref, b_ref, out_ref, bufs, buf_last, sems):
        j = pl.program_id(0)
        slot = lax.rem(j, nbuf)

        @pl.when(j >= nbuf)
        def _drain_oldest():
            pltpu.make_async_copy(
                bufs.at[slot], out_ref.at[:, pl.ds((j - nbuf) * vb, vb)],
                sems.at[slot],
            ).wait()

        acc = lax.dot_general(
            g_ref[...],
            w_ref[...],
            (((1,), (1,)), ((), ())),
            preferred_element_type=jnp.float32,
        )
        out = acc + b_ref[0]

        @pl.when(j < nsteps - 1)
        def _start_full():
            bufs[slot] = out
            pltpu.make_async_copy(
                bufs.at[slot], out_ref.at[:, pl.ds(j * vb, vb)], sems.at[slot]
            ).start()

        @pl.when(j == nsteps - 1)
        def _start_last_and_drain():
            buf_last[...] = out[:, :vb_last]
            pltpu.make_async_copy(
                buf_last,
                out_ref.at[:, pl.ds((nsteps - 1) * vb, vb_last)],
                sems.at[nbuf],
            ).start()
            for k in range(nbuf - 1):
                step = nsteps - nbuf + k
                s = step % nbuf
                pltpu.make_async_copy(
                    bufs.at[s],
                    out_ref.at[:, pl.ds(step * vb, vb)],
                    sems.at[s],
                ).wait()
            pltpu.make_async_copy(
                buf_last,
                out_ref.at[:, pl.ds((nsteps - 1) * vb, vb_last)],
                sems.at[nbuf],
            ).wait()

    call = pl.pallas_call(
        body,
        grid=(nsteps,),
        in_specs=[
            pl.BlockSpec((B, H), lambda j: (0, 0)),
            pl.BlockSpec((vb, H), lambda j: (j, 0)),
            pl.BlockSpec((1, 1, vb), lambda j: (j, 0, 0)),
        ],
        out_specs=pl.BlockSpec(memory_space=pl.ANY),
        out_shape=jax.ShapeDtypeStruct((B, V), jnp.float32),
        scratch_shapes=[
            pltpu.VMEM((nbuf, B, vb), jnp.float32),
            pltpu.VMEM((B, vb_last), jnp.float32),
            pltpu.SemaphoreType.DMA((nbuf + 1,)),
        ],
    )
    b_pad = jnp.pad(head_b, (0, nsteps * vb - V)).reshape(nsteps, 1, vb)
    return call(gathered, head_w, b_pad)


@jax.jit
def kernel(x, embed_table, head_w, head_b):
    gathered = _sc_gather(x, embed_table)
    return _head_matmul(gathered, head_w, head_b, vb=2048, nbuf=4)


# speedup vs baseline: 1.8292x; 1.8292x over previous
import functools

import jax
import jax.numpy as jnp
from jax import lax
from jax.experimental import pallas as pl
from jax.experimental.pallas import tpu as pltpu
from jax.experimental.pallas import tpu_sc as plsc


def _sc_gather(x, embed_table):
    B = x.shape[0]
    H = embed_table.shape[1]
    info = plsc.get_sparse_core_info()
    NW = info.num_cores * info.num_subcores
    assert B % (8 * NW) == 0
    b_per_w = B // NW
    mesh = plsc.VectorSubcoreMesh(core_axis_name="c", subcore_axis_name="s")

    @functools.partial(
        pl.kernel,
        mesh=mesh,
        out_type=jax.ShapeDtypeStruct((B, H), jnp.float32),
        scratch_types=[
            pltpu.VMEM((b_per_w,), jnp.int32),
            pltpu.VMEM((b_per_w, H), jnp.float32),
            pltpu.SemaphoreType.DMA,
        ],
        compiler_params=pltpu.CompilerParams(use_tc_tiling_on_sc=False),
    )
    def gather_kernel(idx_hbm, table_hbm, out_hbm, idx_v, rows_v, sem):
        wid = lax.axis_index("s") * info.num_cores + lax.axis_index("c")
        base = wid * b_per_w
        pltpu.sync_copy(idx_hbm.at[pl.ds(base, b_per_w)], idx_v)
        pltpu.async_copy(table_hbm.at[idx_v], rows_v, sem).wait()
        pltpu.sync_copy(rows_v, out_hbm.at[pl.ds(base, b_per_w)])

    return gather_kernel(x, embed_table)


def _head_matmul_t(gathered, head_w, head_b, vb, nbuf):
    B, H = gathered.shape
    V = head_w.shape[0]
    assert V % vb == 0
    nsteps = V // vb

    def body(g_ref, w_ref, b_ref, out_ref, bufs, sems):
        j = pl.program_id(0)
        slot = lax.rem(j, nbuf)

        @pl.when(j >= nbuf)
        def _drain_oldest():
            pltpu.make_async_copy(
                bufs.at[slot], out_ref.at[pl.ds((j - nbuf) * vb, vb)],
                sems.at[slot],
            ).wait()

        acc = lax.dot_general(
            w_ref[...],
            g_ref[...],
            (((1,), (1,)), ((), ())),
            preferred_element_type=jnp.float32,
        )
        bufs[slot] = acc + b_ref[0]
        pltpu.make_async_copy(
            bufs.at[slot], out_ref.at[pl.ds(j * vb, vb)], sems.at[slot]
        ).start()

        @pl.when(j == nsteps - 1)
        def _drain_rest():
            for k in range(nbuf):
                step = nsteps - nbuf + k
                pltpu.make_async_copy(
                    bufs.at[step % nbuf],
                    out_ref.at[pl.ds(step * vb, vb)],
                    sems.at[step % nbuf],
                ).wait()

    call = pl.pallas_call(
        body,
        grid=(nsteps,),
        in_specs=[
            pl.BlockSpec((B, H), lambda j: (0, 0)),
            pl.BlockSpec((vb, H), lambda j: (j, 0)),
            pl.BlockSpec((1, vb, 1), lambda j: (j, 0, 0)),
        ],
        out_specs=pl.BlockSpec(memory_space=pl.ANY),
        out_shape=jax.ShapeDtypeStruct((V, B), jnp.float32),
        scratch_shapes=[
            pltpu.VMEM((nbuf, vb, B), jnp.float32),
            pltpu.SemaphoreType.DMA((nbuf,)),
        ],
    )
    return call(gathered, head_w, head_b.reshape(nsteps, vb, 1))


@jax.jit
def kernel(x, embed_table, head_w, head_b):
    gathered = _sc_gather(x, embed_table)
    return _head_matmul_t(gathered, head_w, head_b, vb=2000, nbuf=4).T


# speedup vs baseline: 2.5468x; 1.3922x over previous
import functools

import jax
import jax.numpy as jnp
from jax import lax
from jax.experimental import pallas as pl
from jax.experimental.pallas import tpu as pltpu
from jax.experimental.pallas import tpu_sc as plsc


def _sc_gather(x, embed_table):
    B = x.shape[0]
    H = embed_table.shape[1]
    info = plsc.get_sparse_core_info()
    NW = info.num_cores * info.num_subcores
    assert B % (8 * NW) == 0
    b_per_w = B // NW
    mesh = plsc.VectorSubcoreMesh(core_axis_name="c", subcore_axis_name="s")

    @functools.partial(
        pl.kernel,
        mesh=mesh,
        out_type=jax.ShapeDtypeStruct((B, H), jnp.float32),
        scratch_types=[
            pltpu.VMEM((b_per_w,), jnp.int32),
            pltpu.VMEM((b_per_w, H), jnp.float32),
            pltpu.SemaphoreType.DMA,
        ],
        compiler_params=pltpu.CompilerParams(use_tc_tiling_on_sc=False),
    )
    def gather_kernel(idx_hbm, table_hbm, out_hbm, idx_v, rows_v, sem):
        wid = lax.axis_index("s") * info.num_cores + lax.axis_index("c")
        base = wid * b_per_w
        pltpu.sync_copy(idx_hbm.at[pl.ds(base, b_per_w)], idx_v)
        pltpu.async_copy(table_hbm.at[idx_v], rows_v, sem).wait()
        pltpu.sync_copy(rows_v, out_hbm.at[pl.ds(base, b_per_w)])

    return gather_kernel(x, embed_table)


def _head_matmul_t(gathered, head_w, head_b, vb, nbuf):
    B, H = gathered.shape
    V = head_w.shape[0]
    assert V % vb == 0
    nsteps = V // vb

    def body(g_ref, w_ref, b_ref, out_ref, bufs, sems):
        j = pl.program_id(0)
        slot = lax.rem(j, nbuf)

        @pl.when(j >= nbuf)
        def _drain_oldest():
            pltpu.make_async_copy(
                bufs.at[slot], out_ref.at[pl.ds((j - nbuf) * vb, vb)],
                sems.at[slot],
            ).wait()

        acc = lax.dot_general(
            w_ref[...],
            g_ref[...],
            (((1,), (1,)), ((), ())),
            preferred_element_type=jnp.float32,
        )
        bufs[slot] = acc + b_ref[0, 0][:, None]
        pltpu.make_async_copy(
            bufs.at[slot], out_ref.at[pl.ds(j * vb, vb)], sems.at[slot]
        ).start()

        @pl.when(j == nsteps - 1)
        def _drain_rest():
            for k in range(nbuf):
                step = nsteps - nbuf + k
                pltpu.make_async_copy(
                    bufs.at[step % nbuf],
                    out_ref.at[pl.ds(step * vb, vb)],
                    sems.at[step % nbuf],
                ).wait()

    call = pl.pallas_call(
        body,
        grid=(nsteps,),
        in_specs=[
            pl.BlockSpec((B, H), lambda j: (0, 0)),
            pl.BlockSpec((vb, H), lambda j: (j, 0)),
            pl.BlockSpec((1, 1, vb), lambda j: (j, 0, 0)),
        ],
        out_specs=pl.BlockSpec(memory_space=pl.ANY),
        out_shape=jax.ShapeDtypeStruct((V, B), jnp.float32),
        scratch_shapes=[
            pltpu.VMEM((nbuf, vb, B), jnp.float32),
            pltpu.SemaphoreType.DMA((nbuf,)),
        ],
    )
    return call(gathered, head_w, head_b.reshape(nsteps, 1, vb))


@jax.jit
def kernel(x, embed_table, head_w, head_b):
    gathered = _sc_gather(x, embed_table)
    return _head_matmul_t(gathered, head_w, head_b, vb=2000, nbuf=4).T


# speedup vs baseline: 2.5564x; 1.0038x over previous
import functools

import jax
import jax.numpy as jnp
from jax import lax
from jax.experimental import pallas as pl
from jax.experimental.pallas import tpu as pltpu
from jax.experimental.pallas import tpu_sc as plsc


def _sc_gather(x, embed_table):
    B = x.shape[0]
    V, H = embed_table.shape
    table_flat = embed_table.reshape(V * H)
    info = plsc.get_sparse_core_info()
    L = info.num_lanes
    NW = info.num_cores * info.num_subcores
    assert B % (8 * NW) == 0
    b_per_w = B // NW
    n_elem = b_per_w * H
    n_gather = pl.cdiv(n_elem, 128)
    mesh = plsc.VectorSubcoreMesh(core_axis_name="c", subcore_axis_name="s")

    @functools.partial(
        pl.kernel,
        mesh=mesh,
        out_type=jax.ShapeDtypeStruct((H * B,), jnp.float32),
        scratch_types=[
            pltpu.VMEM((b_per_w,), jnp.int32),
            pltpu.VMEM((n_elem,), jnp.int32),
            pltpu.VMEM((n_elem,), jnp.float32),
            pltpu.SemaphoreType.DMA,
            pltpu.SemaphoreType.DMA,
        ],
    )
    def gather_kernel(idx_hbm, table_hbm, out_hbm, xv, gidx, vals, sem, sem2):
        wid = lax.axis_index("s") * info.num_cores + lax.axis_index("c")
        base = wid * b_per_w
        pltpu.sync_copy(idx_hbm.at[pl.ds(base, b_per_w)], xv)
        for c in range(b_per_w // L):
            xc = xv[pl.ds(c * L, L)] * H
            for h in range(H):
                gidx[pl.ds(h * b_per_w + c * L, L)] = xc + h
        for k in range(n_gather):
            pltpu.async_copy(
                table_hbm.at[gidx.at[pl.ds(k * 128, 128)]],
                vals.at[pl.ds(k * 128, 128)],
                sem,
            ).wait()
        for h in range(H):
            pltpu.async_copy(
                vals.at[pl.ds(h * b_per_w, b_per_w)],
                out_hbm.at[pl.ds(h * B + base, b_per_w)],
                sem2,
            ).start()
        for h in range(H):
            pltpu.async_copy(
                vals.at[pl.ds(h * b_per_w, b_per_w)],
                out_hbm.at[pl.ds(h * B + base, b_per_w)],
                sem2,
            ).wait()

    return gather_kernel(x, table_flat).reshape(H, B)


def _head_matmul_t(gathered, head_w, head_b, vb, nbuf):
    H, B = gathered.shape
    V = head_w.shape[0]
    assert V % vb == 0
    nsteps = V // vb

    def body(g_ref, w_ref, b_ref, out_ref, bufs, sems):
        j = pl.program_id(0)
        slot = lax.rem(j, nbuf)

        @pl.when(j >= nbuf)
        def _drain_oldest():
            pltpu.make_async_copy(
                bufs.at[slot], out_ref.at[pl.ds((j - nbuf) * vb, vb)],
                sems.at[slot],
            ).wait()

        acc = lax.dot_general(
            w_ref[...],
            g_ref[...],
            (((1,), (0,)), ((), ())),
            preferred_element_type=jnp.float32,
        )
        bufs[slot] = acc + b_ref[0, 0][:, None]
        pltpu.make_async_copy(
            bufs.at[slot], out_ref.at[pl.ds(j * vb, vb)], sems.at[slot]
        ).start()

        @pl.when(j == nsteps - 1)
        def _drain_rest():
            for k in range(nbuf):
                step = nsteps - nbuf + k
                pltpu.make_async_copy(
                    bufs.at[step % nbuf],
                    out_ref.at[pl.ds(step * vb, vb)],
                    sems.at[step % nbuf],
                ).wait()

    call = pl.pallas_call(
        body,
        grid=(nsteps,),
        in_specs=[
            pl.BlockSpec((H, B), lambda j: (0, 0)),
            pl.BlockSpec((vb, H), lambda j: (j, 0)),
            pl.BlockSpec((1, 1, vb), lambda j: (j, 0, 0)),
        ],
        out_specs=pl.BlockSpec(memory_space=pl.ANY),
        out_shape=jax.ShapeDtypeStruct((V, B), jnp.float32),
        scratch_shapes=[
            pltpu.VMEM((nbuf, vb, B), jnp.float32),
            pltpu.SemaphoreType.DMA((nbuf,)),
        ],
    )
    return call(gathered, head_w, head_b.reshape(nsteps, 1, vb))


@jax.jit
def kernel(x, embed_table, head_w, head_b):
    gathered = _sc_gather(x, embed_table)
    return _head_matmul_t(gathered, head_w, head_b, vb=2000, nbuf=4).T


# speedup vs baseline: 3.7144x; 1.4530x over previous
import functools

import jax
import jax.numpy as jnp
from jax import lax
from jax.experimental import pallas as pl
from jax.experimental.pallas import tpu as pltpu
from jax.experimental.pallas import tpu_sc as plsc


def _sc_gather(x, embed_table):
    B = x.shape[0]
    V, H = embed_table.shape
    table_flat = embed_table.T.reshape(H * V)
    info = plsc.get_sparse_core_info()
    L = info.num_lanes
    NW = info.num_cores * info.num_subcores
    assert B % (8 * NW) == 0
    b_per_w = B // NW
    n_elem = b_per_w * H
    n_gather = pl.cdiv(n_elem, 128)
    mesh = plsc.VectorSubcoreMesh(core_axis_name="c", subcore_axis_name="s")

    @functools.partial(
        pl.kernel,
        mesh=mesh,
        out_type=jax.ShapeDtypeStruct((H * B,), jnp.float32),
        scratch_types=[
            pltpu.VMEM((b_per_w,), jnp.int32),
            pltpu.VMEM((n_elem,), jnp.int32),
            pltpu.VMEM((n_elem,), jnp.float32),
            pltpu.SemaphoreType.DMA,
            pltpu.SemaphoreType.DMA,
        ],
    )
    def gather_kernel(idx_hbm, table_hbm, out_hbm, xv, gidx, vals, sem, sem2):
        wid = lax.axis_index("s") * info.num_cores + lax.axis_index("c")
        base = wid * b_per_w
        pltpu.sync_copy(idx_hbm.at[pl.ds(base, b_per_w)], xv)
        for c in range(b_per_w // L):
            xc = xv[pl.ds(c * L, L)]
            for h in range(H):
                gidx[pl.ds(h * b_per_w + c * L, L)] = xc + h * V
        for k in range(n_gather):
            pltpu.async_copy(
                table_hbm.at[gidx.at[pl.ds(k * 128, 128)]],
                vals.at[pl.ds(k * 128, 128)],
                sem,
            ).wait()
        for h in range(H):
            pltpu.async_copy(
                vals.at[pl.ds(h * b_per_w, b_per_w)],
                out_hbm.at[pl.ds(h * B + base, b_per_w)],
                sem2,
            ).start()
        for h in range(H):
            pltpu.async_copy(
                vals.at[pl.ds(h * b_per_w, b_per_w)],
                out_hbm.at[pl.ds(h * B + base, b_per_w)],
                sem2,
            ).wait()

    return gather_kernel(x, table_flat).reshape(H, B)


def _head_matmul_t(gathered, head_w, head_b, vb, nbuf):
    H, B = gathered.shape
    V = head_w.shape[0]
    nsteps = pl.cdiv(V, vb)
    vb_last = V - (nsteps - 1) * vb

    def body(g_ref, w_ref, b_ref, out_ref, bufs, sems):
        j = pl.program_id(0)
        slot = lax.rem(j, nbuf)

        @pl.when(j >= nbuf)
        def _drain_oldest():
            pltpu.make_async_copy(
                bufs.at[slot], out_ref.at[pl.ds((j - nbuf) * vb, vb)],
                sems.at[slot],
            ).wait()

        acc = lax.dot_general(
            w_ref[...],
            g_ref[...],
            (((0,), (0,)), ((), ())),
            preferred_element_type=jnp.float32,
        )
        bufs[slot] = acc + b_ref[0, 0][:, None]

        @pl.when(j < nsteps - 1)
        def _start_full():
            pltpu.make_async_copy(
                bufs.at[slot], out_ref.at[pl.ds(j * vb, vb)], sems.at[slot]
            ).start()

        @pl.when(j == nsteps - 1)
        def _start_last():
            pltpu.make_async_copy(
                bufs.at[slot, pl.ds(0, vb_last)],
                out_ref.at[pl.ds((nsteps - 1) * vb, vb_last)],
                sems.at[slot],
            ).start()

        @pl.when(j == nsteps - 1)
        def _drain_rest():
            for k in range(nbuf):
                step = nsteps - nbuf + k
                width = vb_last if step == nsteps - 1 else vb
                pltpu.make_async_copy(
                    bufs.at[step % nbuf, pl.ds(0, width)],
                    out_ref.at[pl.ds(step * vb, width)],
                    sems.at[step % nbuf],
                ).wait()

    call = pl.pallas_call(
        body,
        grid=(nsteps,),
        in_specs=[
            pl.BlockSpec((H, B), lambda j: (0, 0)),
            pl.BlockSpec((H, vb), lambda j: (0, j)),
            pl.BlockSpec((1, 1, vb), lambda j: (j, 0, 0)),
        ],
        out_specs=pl.BlockSpec(memory_space=pl.ANY),
        out_shape=jax.ShapeDtypeStruct((V, B), jnp.float32),
        scratch_shapes=[
            pltpu.VMEM((nbuf, vb, B), jnp.float32),
            pltpu.SemaphoreType.DMA((nbuf,)),
        ],
    )
    b_pad = jnp.pad(head_b, (0, nsteps * vb - V)).reshape(nsteps, 1, vb)
    return call(gathered, head_w.T, b_pad)


@jax.jit
def kernel(x, embed_table, head_w, head_b):
    gathered = _sc_gather(x, embed_table)
    return _head_matmul_t(gathered, head_w, head_b, vb=2048, nbuf=4).T


# speedup vs baseline: 3.7144x; 1.0000x over previous
import functools

import jax
import jax.numpy as jnp
from jax import lax
from jax.experimental import pallas as pl
from jax.experimental.pallas import tpu as pltpu
from jax.experimental.pallas import tpu_sc as plsc


def _sc_gather(x, embed_table):
    B = x.shape[0]
    V, H = embed_table.shape
    table_flat = embed_table.T.reshape(H * V)
    info = plsc.get_sparse_core_info()
    L = info.num_lanes
    NW = info.num_cores * info.num_subcores
    assert B % (8 * NW) == 0
    b_per_w = B // NW
    n_elem = b_per_w * H
    n_gather = pl.cdiv(n_elem, 128)
    mesh = plsc.VectorSubcoreMesh(core_axis_name="c", subcore_axis_name="s")

    @functools.partial(
        pl.kernel,
        mesh=mesh,
        out_type=jax.ShapeDtypeStruct((H * B,), jnp.float32),
        scratch_types=[
            pltpu.VMEM((b_per_w,), jnp.int32),
            pltpu.VMEM((n_elem,), jnp.int32),
            pltpu.VMEM((n_elem,), jnp.float32),
            pltpu.SemaphoreType.DMA,
            pltpu.SemaphoreType.DMA,
        ],
    )
    def gather_kernel(idx_hbm, table_hbm, out_hbm, xv, gidx, vals, sem, sem2):
        wid = lax.axis_index("s") * info.num_cores + lax.axis_index("c")
        base = wid * b_per_w
        pltpu.sync_copy(idx_hbm.at[pl.ds(base, b_per_w)], xv)
        for c in range(b_per_w // L):
            xc = xv[pl.ds(c * L, L)]
            for h in range(H):
                gidx[pl.ds(h * b_per_w + c * L, L)] = xc + h * V
        for k in range(n_gather):
            pltpu.async_copy(
                table_hbm.at[gidx.at[pl.ds(k * 128, 128)]],
                vals.at[pl.ds(k * 128, 128)],
                sem,
            ).wait()
        for h in range(H):
            pltpu.async_copy(
                vals.at[pl.ds(h * b_per_w, b_per_w)],
                out_hbm.at[pl.ds(h * B + base, b_per_w)],
                sem2,
            ).start()
        for h in range(H):
            pltpu.async_copy(
                vals.at[pl.ds(h * b_per_w, b_per_w)],
                out_hbm.at[pl.ds(h * B + base, b_per_w)],
                sem2,
            ).wait()

    return gather_kernel(x, table_flat).reshape(H, B)


def _head_matmul_t(gathered, head_w, head_b, vb, nbuf):
    H, B = gathered.shape
    V = head_w.shape[0]
    nsteps = pl.cdiv(V, vb)
    vb_last = V - (nsteps - 1) * vb

    def body(g_ref, w_ref, b_ref, out_ref, bufs, sems):
        j = pl.program_id(0)
        slot = lax.rem(j, nbuf)

        @pl.when(j >= nbuf)
        def _drain_oldest():
            pltpu.make_async_copy(
                bufs.at[slot], out_ref.at[pl.ds((j - nbuf) * vb, vb)],
                sems.at[slot],
            ).wait()

        acc = lax.dot_general(
            w_ref[...],
            g_ref[...],
            (((0,), (0,)), ((), ())),
            preferred_element_type=jnp.float32,
        )
        bufs[slot] = acc + b_ref[0, 0][:, None]

        @pl.when(j < nsteps - 1)
        def _start_full():
            pltpu.make_async_copy(
                bufs.at[slot], out_ref.at[pl.ds(j * vb, vb)], sems.at[slot]
            ).start()

        @pl.when(j == nsteps - 1)
        def _start_last():
            pltpu.make_async_copy(
                bufs.at[slot, pl.ds(0, vb_last)],
                out_ref.at[pl.ds((nsteps - 1) * vb, vb_last)],
                sems.at[slot],
            ).start()

        @pl.when(j == nsteps - 1)
        def _drain_rest():
            for k in range(nbuf):
                step = nsteps - nbuf + k
                width = vb_last if step == nsteps - 1 else vb
                pltpu.make_async_copy(
                    bufs.at[step % nbuf, pl.ds(0, width)],
                    out_ref.at[pl.ds(step * vb, width)],
                    sems.at[step % nbuf],
                ).wait()

    call = pl.pallas_call(
        body,
        grid=(nsteps,),
        in_specs=[
            pl.BlockSpec((H, B), lambda j: (0, 0)),
            pl.BlockSpec((H, vb), lambda j: (0, j)),
            pl.BlockSpec((1, 1, vb), lambda j: (j, 0, 0)),
        ],
        out_specs=pl.BlockSpec(memory_space=pl.ANY),
        out_shape=jax.ShapeDtypeStruct((V, B), jnp.float32),
        scratch_shapes=[
            pltpu.VMEM((nbuf, vb, B), jnp.float32),
            pltpu.SemaphoreType.DMA((nbuf,)),
        ],
    )
    b_pad = jnp.pad(head_b, (0, nsteps * vb - V)).reshape(nsteps, 1, vb)
    return call(gathered, head_w.T, b_pad)


@jax.jit
def kernel(x, embed_table, head_w, head_b):
    gathered = _sc_gather(x, embed_table)
    return _head_matmul_t(gathered, head_w, head_b, vb=2048, nbuf=6).T
